# Initial kernel scaffold; baseline (speedup 1.0000x reference)
import jax
import jax.numpy as jnp
from jax.experimental import pallas as pl


def kernel(encoded_tokens, pos_table):
    raise NotImplementedError("write your pallas kernel here")


# speedup vs baseline: 1.8006x; 1.8006x over previous
import jax
import jax.numpy as jnp
from jax.experimental import pallas as pl


_TBLK = 2048


def _add_kernel(tok_ref, tab_ref, out_ref):
    out_ref[0] = tok_ref[0] + tab_ref[...]


def kernel(encoded_tokens, pos_table):
    batch, num_tokens, embed_dim = encoded_tokens.shape
    grid = (num_tokens // _TBLK, batch)
    return pl.pallas_call(
        _add_kernel,
        grid=grid,
        in_specs=[
            pl.BlockSpec((1, _TBLK, embed_dim), lambda t, b: (b, t, 0)),
            pl.BlockSpec((_TBLK, embed_dim), lambda t, b: (t, 0)),
        ],
        out_specs=pl.BlockSpec((1, _TBLK, embed_dim), lambda t, b: (b, t, 0)),
        out_shape=jax.ShapeDtypeStruct(encoded_tokens.shape, encoded_tokens.dtype),
    )(encoded_tokens, pos_table)
